# baseline (speedup 1.0000x reference)
import functools

import jax
import jax.numpy as jnp
from jax import lax
from jax.experimental import pallas as pl
from jax.experimental.pallas import tpu as pltpu
from jax.experimental.pallas import tpu_sc as plsc

VOCAB = 1000000
MAX_LEN = 200
EMBED = 64
BATCH = 1024
SEQ = 200

NC = 2
NS = 16
NW = NC * NS
SEQ_PER_W = BATCH // NW
HALF = SEQ // 2


def _sc_body(x_hbm, table_hbm, pos_hbm, out_hbm, idx_s, rows_s, pos_v, sem):
    wid = lax.axis_index("s") * NC + lax.axis_index("c")

    pltpu.sync_copy(pos_hbm, pos_v)

    def seq_body(i, carry):
        b = wid * SEQ_PER_W + i
        pltpu.sync_copy(x_hbm.at[b], idx_s)
        cp0 = pltpu.async_copy(table_hbm.at[idx_s.at[0]],
                               rows_s.at[pl.ds(0, HALF)], sem)
        cp1 = pltpu.async_copy(table_hbm.at[idx_s.at[1]],
                               rows_s.at[pl.ds(HALF, HALF)], sem)
        cp0.wait()
        cp1.wait()

        def add_row(r, c2):
            for c in range(EMBED // 16):
                sl = pl.ds(c * 16, 16)
                rows_s[r, sl] = rows_s[r, sl] + pos_v[r, sl]
            return c2

        lax.fori_loop(0, SEQ, add_row, 0, unroll=2)

        pltpu.sync_copy(rows_s, out_hbm.at[b])
        return carry

    lax.fori_loop(0, SEQ_PER_W, seq_body, 0)


def kernel(x, token_table, pos_table):
    x3 = x.reshape(BATCH, 2, HALF)
    mesh = plsc.VectorSubcoreMesh(core_axis_name="c", subcore_axis_name="s")
    k = functools.partial(
        pl.kernel,
        mesh=mesh,
        out_type=jax.ShapeDtypeStruct((BATCH, SEQ, EMBED), jnp.float32),
        scratch_types=[
            pltpu.VMEM((2, HALF), jnp.int32),
            pltpu.VMEM((SEQ, EMBED), jnp.float32),
            pltpu.VMEM((MAX_LEN, EMBED), jnp.float32),
            pltpu.SemaphoreType.DMA,
        ],
        compiler_params=pltpu.CompilerParams(use_tc_tiling_on_sc=False),
    )(_sc_body)
    return k(x3, token_table, pos_table)


# speedup vs baseline: 1.0605x; 1.0605x over previous
import functools

import jax
import jax.numpy as jnp
from jax import lax
from jax.experimental import pallas as pl
from jax.experimental.pallas import tpu as pltpu
from jax.experimental.pallas import tpu_sc as plsc

VOCAB = 1000000
MAX_LEN = 200
EMBED = 64
BATCH = 1024
SEQ = 200

NC = 2
NS = 16
NW = NC * NS
SEQ_PER_W = BATCH // NW
HALF = SEQ // 2
NGRP = EMBED // 16


def _sc_body(x_hbm, table_hbm, pos_hbm, out_hbm,
             idx_all, rows, pos_v, sem_g0, sem_g1, sem_o0, sem_o1):
    wid = lax.axis_index("s") * NC + lax.axis_index("c")
    base = wid * SEQ_PER_W
    sem_g = (sem_g0, sem_g1)
    sem_o = (sem_o0, sem_o1)

    pltpu.sync_copy(pos_hbm, pos_v)
    pltpu.sync_copy(x_hbm.at[pl.ds(base, SEQ_PER_W)], idx_all)

    def start_gathers(s, sl):
        for h in range(2):
            pltpu.async_copy(table_hbm.at[idx_all.at[s, h]],
                             rows.at[sl, pl.ds(h * HALF, HALF)], sem_g[sl])

    def wait_gathers(s, sl):
        for h in range(2):
            pltpu.make_async_copy(table_hbm.at[idx_all.at[s, h]],
                                  rows.at[sl, pl.ds(h * HALF, HALF)],
                                  sem_g[sl]).wait()

    def out_store(s, sl):
        return pltpu.make_async_copy(rows.at[sl], out_hbm.at[base + s],
                                     sem_o[sl])

    def handle(s, sl):
        nsl = 1 - sl
        @pl.when(s >= 1)
        def _():
            out_store(s - 1, nsl).wait()

        @pl.when(s <= SEQ_PER_W - 2)
        def _():
            start_gathers(s + 1, nsl)

        wait_gathers(s, sl)

        def add_row(r, c2):
            for c in range(NGRP):
                cs = pl.ds(c * 16, 16)
                rows[sl, r, cs] = rows[sl, r, cs] + pos_v[r, cs]
            return c2

        lax.fori_loop(0, SEQ, add_row, 0, unroll=4)
        out_store(s, sl).start()

    start_gathers(0, 0)

    def pair(p, carry):
        handle(2 * p, 0)
        handle(2 * p + 1, 1)
        return carry

    lax.fori_loop(0, SEQ_PER_W // 2, pair, 0)
    out_store(SEQ_PER_W - 1, 1).wait()


def kernel(x, token_table, pos_table):
    x3 = x.reshape(BATCH, 2, HALF)
    mesh = plsc.VectorSubcoreMesh(core_axis_name="c", subcore_axis_name="s")
    k = functools.partial(
        pl.kernel,
        mesh=mesh,
        out_type=jax.ShapeDtypeStruct((BATCH, SEQ, EMBED), jnp.float32),
        scratch_types=[
            pltpu.VMEM((SEQ_PER_W, 2, HALF), jnp.int32),
            pltpu.VMEM((2, SEQ, EMBED), jnp.float32),
            pltpu.VMEM((MAX_LEN, EMBED), jnp.float32),
            pltpu.SemaphoreType.DMA,
            pltpu.SemaphoreType.DMA,
            pltpu.SemaphoreType.DMA,
            pltpu.SemaphoreType.DMA,
        ],
        compiler_params=pltpu.CompilerParams(use_tc_tiling_on_sc=False),
    )(_sc_body)
    return k(x3, token_table, pos_table)
